# baseline (speedup 1.0000x reference)
import jax
import jax.numpy as jnp
from jax import lax
from jax.experimental import pallas as pl
from jax.experimental.pallas import tpu as pltpu
from jax.experimental.pallas import tpu_sc as plsc

N = 10000
E = 320000
D = 128
H = 8
C = 16
HC = H * C
NPAD = 10240
ROWW = 144
NC = 2
NS = 16
NW = NC * NS
EPW = E // NW
CH = 128
NCH = 80
PADE = NCH * CH - EPW
BN = 2000
NB = N // BN
RPT = NPAD // NS

_HIGH = lax.Precision.HIGHEST


def _tc12_body(x_ref, wq_ref, bq_ref, ab_ref, p_ref, e8_ref, t_ref,
               q_s, qa_s, gm_s):
    p = pl.program_id(0)
    i = pl.program_id(1)

    @pl.when(p == 0)
    def _():
        xq = jnp.dot(x_ref[...], wq_ref[...], precision=_HIGH) + bq_ref[...]
        q_s[pl.ds(i * BN, BN), :] = xq
        qa = jnp.dot(xq, ab_ref[...], precision=_HIGH)
        qa_s[pl.ds(i * BN, BN), :] = qa
        bm = jnp.max(qa, axis=0, keepdims=True)

        @pl.when(i == 0)
        def _():
            gm_s[...] = bm

        @pl.when(i != 0)
        def _():
            gm_s[...] = jnp.maximum(gm_s[...], bm)

    @pl.when(p == 1)
    def _():
        qz = jnp.exp(qa_s[pl.ds(i * BN, BN), :] - gm_s[...])
        qzrep = jnp.dot(qz, p_ref[...], precision=_HIGH)
        t_ref[:, :HC] = q_s[pl.ds(i * BN, BN), :] * qzrep
        t_ref[:, HC:ROWW] = jnp.dot(qz, e8_ref[...], precision=_HIGH)


def _tc3_body(a0_ref, a1_ref, b0_ref, b1_ref, p_ref, o_ref):
    num = a0_ref[0] + a1_ref[0]
    den = b0_ref[0, :, :H] + b1_ref[0, :, :H]
    dexp = jnp.dot(den, p_ref[...], precision=_HIGH)
    o_ref[...] = jnp.maximum(num / (dexp + 1e-16), 0.0)


def _sc_body(t_hbm, pair_hbm, z_hbm, outa_hbm, outb_hbm,
             ib0, ib1, rows0, rows1, accum, sem0, sem1):
    c = lax.axis_index("c")
    s = lax.axis_index("s")
    wid = c * NS + s
    rb = pl.multiple_of(s * RPT, 8)

    def ldp(j, ib):
        pltpu.sync_copy(pair_hbm.at[wid, j], ib)

    def start(ib, buf, sem):
        pltpu.async_copy(t_hbm.at[ib.at[0]], buf, sem)

    def wait(buf, sem):
        pltpu.make_async_copy(t_hbm.at[ib0.at[0]], buf, sem).wait()

    def scat(ib, buf):
        pltpu.sync_copy(buf, accum.at[ib.at[1]], add=True)

    pltpu.sync_copy(z_hbm.at[pl.ds(rb, RPT)], accum.at[pl.ds(rb, RPT)])
    ldp(0, ib0)
    start(ib0, rows0, sem0)
    plsc.subcore_barrier()

    def body(jj, carry):
        j = 2 * jj
        ldp(j + 1, ib1)
        start(ib1, rows1, sem1)
        wait(rows0, sem0)
        scat(ib0, rows0)
        ldp(j + 2, ib0)
        start(ib0, rows0, sem0)
        wait(rows1, sem1)
        scat(ib1, rows1)
        return carry

    lax.fori_loop(0, (NCH - 2) // 2, body, 0, unroll=False)

    ldp(NCH - 1, ib1)
    start(ib1, rows1, sem1)
    wait(rows0, sem0)
    scat(ib0, rows0)
    wait(rows1, sem1)
    scat(ib1, rows1)

    plsc.subcore_barrier()

    pltpu.sync_copy(accum.at[pl.ds(rb, RPT), pl.ds(0, HC)],
                    outa_hbm.at[c, pl.ds(rb, RPT)])
    pltpu.sync_copy(accum.at[pl.ds(rb, RPT), pl.ds(HC, ROWW - HC)],
                    outb_hbm.at[c, pl.ds(rb, RPT)])


def kernel(x, Wq, bq, Wk, bk, A, edge_index):
    del Wk, bk
    f32 = jnp.float32
    ab = (A.T[:, :, None] * jnp.eye(H, dtype=f32)[:, None, :]).reshape(HC, H)
    p_exp = jnp.kron(jnp.eye(H, dtype=f32), jnp.ones((1, C), f32))
    e8 = jnp.concatenate([jnp.eye(H, dtype=f32),
                          jnp.zeros((H, ROWW - HC - H), f32)], axis=1)
    bq2 = bq.reshape(1, HC)
    dpad = jnp.broadcast_to(N + jnp.arange(PADE, dtype=jnp.int32), (NW, PADE))
    srcp = jnp.concatenate(
        [edge_index[0].reshape(NW, EPW),
         jnp.zeros((NW, PADE), jnp.int32)], axis=1).reshape(NW, NCH, CH)
    dstp = jnp.concatenate(
        [edge_index[1].reshape(NW, EPW), dpad], axis=1).reshape(NW, NCH, CH)
    pair = jnp.stack([srcp, dstp], axis=2)
    zrows = jnp.zeros((NPAD, ROWW), f32)

    t_tab = pl.pallas_call(
        _tc12_body,
        grid=(2, NB),
        in_specs=[
            pl.BlockSpec((BN, D), lambda p, i: (i * (1 - p), 0)),
            pl.BlockSpec((D, HC), lambda p, i: (0, 0)),
            pl.BlockSpec((1, HC), lambda p, i: (0, 0)),
            pl.BlockSpec((HC, H), lambda p, i: (0, 0)),
            pl.BlockSpec((H, HC), lambda p, i: (0, 0)),
            pl.BlockSpec((H, ROWW - HC), lambda p, i: (0, 0)),
        ],
        out_specs=pl.BlockSpec((BN, ROWW),
                               lambda p, i: (NB * (1 - p) + i * p, 0)),
        out_shape=jax.ShapeDtypeStruct((N + BN, ROWW), f32),
        scratch_shapes=[
            pltpu.VMEM((N, HC), f32),
            pltpu.VMEM((N, H), f32),
            pltpu.VMEM((1, H), f32),
        ],
    )(x, Wq, bq2, ab, p_exp, e8)

    mesh = plsc.VectorSubcoreMesh(core_axis_name="c", subcore_axis_name="s",
                                  num_cores=NC, num_subcores=NS)
    acca, accb = pl.kernel(
        _sc_body,
        out_type=[jax.ShapeDtypeStruct((NC, NPAD, HC), f32),
                  jax.ShapeDtypeStruct((NC, NPAD, ROWW - HC), f32)],
        mesh=mesh,
        scratch_types=[
            pltpu.VMEM((2, CH), jnp.int32),
            pltpu.VMEM((2, CH), jnp.int32),
            pltpu.VMEM((CH, ROWW), f32),
            pltpu.VMEM((CH, ROWW), f32),
            pltpu.VMEM_SHARED((NPAD, ROWW), f32),
            pltpu.SemaphoreType.DMA,
            pltpu.SemaphoreType.DMA,
        ],
        compiler_params=pltpu.CompilerParams(use_tc_tiling_on_sc=False),
    )(t_tab, pair, zrows)

    BN3 = 2000
    out = pl.pallas_call(
        _tc3_body,
        grid=(N // BN3,),
        in_specs=[
            pl.BlockSpec((1, BN3, HC), lambda i: (0, i, 0)),
            pl.BlockSpec((1, BN3, HC), lambda i: (1, i, 0)),
            pl.BlockSpec((1, BN3, ROWW - HC), lambda i: (0, i, 0)),
            pl.BlockSpec((1, BN3, ROWW - HC), lambda i: (1, i, 0)),
            pl.BlockSpec((H, HC), lambda i: (0, 0)),
        ],
        out_specs=pl.BlockSpec((BN3, HC), lambda i: (i, 0)),
        out_shape=jax.ShapeDtypeStruct((N, HC), f32),
    )(acca, acca, accb, accb, p_exp)

    return out


# speedup vs baseline: 2.1529x; 2.1529x over previous
import jax
import jax.numpy as jnp
from jax import lax
from jax.experimental import pallas as pl
from jax.experimental.pallas import tpu as pltpu
from jax.experimental.pallas import tpu_sc as plsc

N = 10000
E = 320000
D = 128
H = 8
C = 16
HC = H * C
NPAD = 10240
ROWW = 144
NC = 2
NS = 16
NW = NC * NS
EPW = E // NW
CH = 128
NFULL = EPW // CH
TAIL = EPW - NFULL * CH
BN = 2000
NB = N // BN
RPT = NPAD // NS

_HIGH = lax.Precision.HIGHEST


def _tc12_body(x_ref, wq_ref, bq_ref, ab_ref, p_ref, e8_ref, t_ref,
               q_s, qa_s, gm_s):
    p = pl.program_id(0)
    i = pl.program_id(1)

    @pl.when(p == 0)
    def _():
        xq = jnp.dot(x_ref[...], wq_ref[...], precision=_HIGH) + bq_ref[...]
        q_s[pl.ds(i * BN, BN), :] = xq
        qa = jnp.dot(xq, ab_ref[...], precision=_HIGH)
        qa_s[pl.ds(i * BN, BN), :] = qa
        bm = jnp.max(qa, axis=0, keepdims=True)

        @pl.when(i == 0)
        def _():
            gm_s[...] = bm

        @pl.when(i != 0)
        def _():
            gm_s[...] = jnp.maximum(gm_s[...], bm)

    @pl.when(p == 1)
    def _():
        qz = jnp.exp(qa_s[pl.ds(i * BN, BN), :] - gm_s[...])
        qzrep = jnp.dot(qz, p_ref[...], precision=_HIGH)
        t_ref[:, :HC] = q_s[pl.ds(i * BN, BN), :] * qzrep
        t_ref[:, HC:ROWW] = jnp.dot(qz, e8_ref[...], precision=_HIGH)


def _tc3_body(a0_ref, a1_ref, b0_ref, b1_ref, p_ref, o_ref):
    num = a0_ref[0] + a1_ref[0]
    den = b0_ref[0, :, :H] + b1_ref[0, :, :H]
    dexp = jnp.dot(den, p_ref[...], precision=_HIGH)
    o_ref[...] = jnp.maximum(num / (dexp + 1e-16), 0.0)


def _sc_body(t_hbm, ei_hbm, z_hbm, outa_hbm, outb_hbm,
             sidx0, didx0, sidx1, didx1, rows0, rows1, didx_t, accum,
             sem0, sem1):
    c = lax.axis_index("c")
    s = lax.axis_index("s")
    base = pl.multiple_of((c * NS + s) * EPW, 8)
    rb = pl.multiple_of(s * RPT, 8)

    def ldidx(j, sidx, didx):
        eb = pl.multiple_of(base + j * CH, 8)
        pltpu.sync_copy(ei_hbm.at[pl.ds(eb, CH)], sidx)
        pltpu.sync_copy(ei_hbm.at[pl.ds(E + eb, CH)], didx)

    def start(sidx, buf, sem):
        pltpu.async_copy(t_hbm.at[sidx], buf, sem)

    def wait(buf, sem):
        pltpu.make_async_copy(t_hbm.at[sidx0], buf, sem).wait()

    def scat(didx, buf):
        pltpu.sync_copy(buf, accum.at[didx], add=True)

    pltpu.sync_copy(z_hbm.at[pl.ds(rb, RPT)], accum.at[pl.ds(rb, RPT)])
    ldidx(0, sidx0, didx0)
    start(sidx0, rows0, sem0)
    plsc.subcore_barrier()

    def body(jj, carry):
        j = 2 * jj
        ldidx(j + 1, sidx1, didx1)
        start(sidx1, rows1, sem1)
        wait(rows0, sem0)
        scat(didx0, rows0)
        ldidx(j + 2, sidx0, didx0)
        start(sidx0, rows0, sem0)
        wait(rows1, sem1)
        scat(didx1, rows1)
        return carry

    lax.fori_loop(0, (NFULL - 2) // 2, body, 0, unroll=False)

    ldidx(NFULL - 1, sidx1, didx1)
    start(sidx1, rows1, sem1)
    wait(rows0, sem0)
    scat(didx0, rows0)
    wait(rows1, sem1)
    scat(didx1, rows1)

    eb = pl.multiple_of(base + NFULL * CH, 8)
    pltpu.sync_copy(ei_hbm.at[pl.ds(eb, TAIL)], sidx0.at[pl.ds(0, TAIL)])
    pltpu.sync_copy(ei_hbm.at[pl.ds(E + eb, TAIL)], didx_t)
    pltpu.async_copy(t_hbm.at[sidx0.at[pl.ds(0, TAIL)]],
                     rows0.at[pl.ds(0, TAIL)], sem0).wait()
    pltpu.sync_copy(rows0.at[pl.ds(0, TAIL)], accum.at[didx_t], add=True)

    plsc.subcore_barrier()

    pltpu.sync_copy(accum.at[pl.ds(rb, RPT), pl.ds(0, HC)],
                    outa_hbm.at[c, pl.ds(rb, RPT)])
    pltpu.sync_copy(accum.at[pl.ds(rb, RPT), pl.ds(HC, ROWW - HC)],
                    outb_hbm.at[c, pl.ds(rb, RPT)])


def kernel(x, Wq, bq, Wk, bk, A, edge_index):
    del Wk, bk
    f32 = jnp.float32
    ab = (A.T[:, :, None] * jnp.eye(H, dtype=f32)[:, None, :]).reshape(HC, H)
    p_exp = jnp.kron(jnp.eye(H, dtype=f32), jnp.ones((1, C), f32))
    e8 = jnp.concatenate([jnp.eye(H, dtype=f32),
                          jnp.zeros((H, ROWW - HC - H), f32)], axis=1)
    bq2 = bq.reshape(1, HC)
    eflat = edge_index.reshape(2 * E)
    zrows = jnp.zeros((NPAD, ROWW), f32)

    t_tab = pl.pallas_call(
        _tc12_body,
        grid=(2, NB),
        in_specs=[
            pl.BlockSpec((BN, D), lambda p, i: (i * (1 - p), 0)),
            pl.BlockSpec((D, HC), lambda p, i: (0, 0)),
            pl.BlockSpec((1, HC), lambda p, i: (0, 0)),
            pl.BlockSpec((HC, H), lambda p, i: (0, 0)),
            pl.BlockSpec((H, HC), lambda p, i: (0, 0)),
            pl.BlockSpec((H, ROWW - HC), lambda p, i: (0, 0)),
        ],
        out_specs=pl.BlockSpec((BN, ROWW),
                               lambda p, i: (NB * (1 - p) + i * p, 0)),
        out_shape=jax.ShapeDtypeStruct((N + BN, ROWW), f32),
        scratch_shapes=[
            pltpu.VMEM((N, HC), f32),
            pltpu.VMEM((N, H), f32),
            pltpu.VMEM((1, H), f32),
        ],
    )(x, Wq, bq2, ab, p_exp, e8)

    mesh = plsc.VectorSubcoreMesh(core_axis_name="c", subcore_axis_name="s",
                                  num_cores=NC, num_subcores=NS)
    acca, accb = pl.kernel(
        _sc_body,
        out_type=[jax.ShapeDtypeStruct((NC, NPAD, HC), f32),
                  jax.ShapeDtypeStruct((NC, NPAD, ROWW - HC), f32)],
        mesh=mesh,
        scratch_types=[
            pltpu.VMEM((CH,), jnp.int32),
            pltpu.VMEM((CH,), jnp.int32),
            pltpu.VMEM((CH,), jnp.int32),
            pltpu.VMEM((CH,), jnp.int32),
            pltpu.VMEM((CH, ROWW), f32),
            pltpu.VMEM((CH, ROWW), f32),
            pltpu.VMEM((TAIL,), jnp.int32),
            pltpu.VMEM_SHARED((NPAD, ROWW), f32),
            pltpu.SemaphoreType.DMA,
            pltpu.SemaphoreType.DMA,
        ],
        compiler_params=pltpu.CompilerParams(use_tc_tiling_on_sc=False),
    )(t_tab, eflat, zrows)

    BN3 = 2000
    out = pl.pallas_call(
        _tc3_body,
        grid=(N // BN3,),
        in_specs=[
            pl.BlockSpec((1, BN3, HC), lambda i: (0, i, 0)),
            pl.BlockSpec((1, BN3, HC), lambda i: (1, i, 0)),
            pl.BlockSpec((1, BN3, ROWW - HC), lambda i: (0, i, 0)),
            pl.BlockSpec((1, BN3, ROWW - HC), lambda i: (1, i, 0)),
            pl.BlockSpec((H, HC), lambda i: (0, 0)),
        ],
        out_specs=pl.BlockSpec((BN3, HC), lambda i: (i, 0)),
        out_shape=jax.ShapeDtypeStruct((N, HC), f32),
    )(acca, acca, accb, accb, p_exp)

    return out
